# baseline (speedup 1.0000x reference)
import jax
import jax.numpy as jnp
from jax import lax
from jax.experimental import pallas as pl
from jax.experimental.pallas import tpu as pltpu
from jax.experimental.pallas import tpu_sc as plsc

B = 16384
ED = 64
RD = 32
NE = 1000000
NW = 32
PASS = 128
NPASS = B // (NW * PASS)
CH = 16
NCH = PASS // CH


def _sc_body(head_hbm, rel_hbm, tail_hbm, entf_hbm, relemb_hbm, projt_hbm,
             out_hbm, hadr, tadr, hraw, traw, rebuf, obuf, pb0, pb1,
             hidx, tidx, ridx, sem_g, sem_p0, sem_p1):
  wid = lax.axis_index("s") * 2 + lax.axis_index("c")
  iota = lax.iota(jnp.int32, 16)
  kbase = [(iota + c * 16) * NE for c in range(ED // 16)]

  def one_pass(p, carry):
    brow = wid * NPASS + p
    base = brow * PASS

    pltpu.sync_copy(head_hbm.at[pl.ds(brow, 1)], hidx)
    pltpu.sync_copy(tail_hbm.at[pl.ds(brow, 1)], tidx)
    pltpu.sync_copy(rel_hbm.at[pl.ds(brow, 1)], ridx)

    def bld(bb, carry2):
      hv = hidx[0, pl.ds(bb * 16, 16)]
      tv = tidx[0, pl.ds(bb * 16, 16)]
      for j in range(16):
        hs = jnp.broadcast_to(hv[j], (16,))
        ts = jnp.broadcast_to(tv[j], (16,))
        off = (bb * 16 + j) * ED
        for c in range(ED // 16):
          hadr[pl.ds(off + c * 16, 16)] = kbase[c] + hs
          tadr[pl.ds(off + c * 16, 16)] = kbase[c] + ts
      return carry2
    lax.fori_loop(0, PASS // 16, bld, 0)

    g1 = pltpu.make_async_copy(entf_hbm.at[hadr], hraw, sem_g)
    g2 = pltpu.make_async_copy(entf_hbm.at[tadr], traw, sem_g)
    g3 = pltpu.make_async_copy(relemb_hbm.at[ridx.at[0]], rebuf, sem_g)
    g1.start(); g2.start(); g3.start()

    def p_desc(c, buf, sem):
      return pltpu.make_async_copy(
          projt_hbm.at[ridx.at[0, pl.ds(c * CH, CH)]], buf, sem)

    p_desc(0, pb0, sem_p0).start()
    p_desc(1, pb1, sem_p1).start()

    g1.wait(); g2.wait(); g3.wait()

    def dsub(i, carry2):
      col = i * 16
      hraw[pl.ds(col, 16)] = hraw[pl.ds(col, 16)] - traw[pl.ds(col, 16)]
      return carry2
    lax.fori_loop(0, PASS * ED // 16, dsub, 0)

    def ring(it, carry2):
      for bb, (buf, sem) in enumerate(((pb0, sem_p0), (pb1, sem_p1))):
        c = it * 2 + bb
        p_desc(c, buf, sem).wait()

        def triple(s, carry3):
          b = c * CH + s
          a0 = rebuf[b, pl.ds(0, 16)]
          a1 = rebuf[b, pl.ds(16, 16)]
          for kk in range(ED // 16):
            dv = hraw[pl.ds(b * ED + kk * 16, 16)]
            for j in range(16):
              k = kk * 16 + j
              bc = jnp.broadcast_to(dv[j], (16,))
              p0 = buf[s, pl.ds(k * RD, 16)]
              p1 = buf[s, pl.ds(k * RD + 16, 16)]
              a0 = a0 + bc * p0
              a1 = a1 + bc * p1
          obuf[b, pl.ds(0, 16)] = a0
          obuf[b, pl.ds(16, 16)] = a1
          return carry3
        lax.fori_loop(0, CH, triple, 0)

        nxt = c + 2

        @pl.when(nxt < NCH)
        def _():
          p_desc(nxt, buf, sem).start()
      return carry2
    lax.fori_loop(0, NCH // 2, ring, 0)

    pltpu.sync_copy(obuf, out_hbm.at[pl.ds(base, PASS)])
    return carry
  lax.fori_loop(0, NPASS, one_pass, 0)


def _tc_norm_body(x_ref, o_ref):
  x = x_ref[...]
  o_ref[...] = jnp.sqrt(jnp.sum(x * x, axis=1))


def kernel(head, relation, tail, entity_table, relation_table, proj_table):
  head2 = head.reshape(128, 128).astype(jnp.int32)
  rel2 = relation.reshape(128, 128).astype(jnp.int32)
  tail2 = tail.reshape(128, 128).astype(jnp.int32)
  entf = entity_table.T.reshape(-1)
  projt = proj_table.reshape(-1, RD, ED).transpose(0, 2, 1).reshape(-1, RD * ED)

  sc = pl.kernel(
      _sc_body,
      out_type=jax.ShapeDtypeStruct((B, RD), jnp.float32),
      mesh=plsc.VectorSubcoreMesh(core_axis_name="c", subcore_axis_name="s"),
      compiler_params=pltpu.CompilerParams(use_tc_tiling_on_sc=False),
      scratch_types=[
          pltpu.VMEM((PASS * ED,), jnp.int32),
          pltpu.VMEM((PASS * ED,), jnp.int32),
          pltpu.VMEM((PASS * ED,), jnp.float32),
          pltpu.VMEM((PASS * ED,), jnp.float32),
          pltpu.VMEM((PASS, RD), jnp.float32),
          pltpu.VMEM((PASS, RD), jnp.float32),
          pltpu.VMEM((CH, RD * ED), jnp.float32),
          pltpu.VMEM((CH, RD * ED), jnp.float32),
          pltpu.VMEM((1, PASS), jnp.int32),
          pltpu.VMEM((1, PASS), jnp.int32),
          pltpu.VMEM((1, PASS), jnp.int32),
          pltpu.SemaphoreType.DMA,
          pltpu.SemaphoreType.DMA,
          pltpu.SemaphoreType.DMA,
      ],
  )
  diff = sc(head2, rel2, tail2, entf, relation_table, projt)

  out = pl.pallas_call(
      _tc_norm_body,
      grid=(16,),
      in_specs=[pl.BlockSpec((B // 16, RD), lambda i: (i, 0))],
      out_specs=pl.BlockSpec((B // 16,), lambda i: (i,)),
      out_shape=jax.ShapeDtypeStruct((B,), jnp.float32),
  )(diff)
  return out


# speedup vs baseline: 1.0005x; 1.0005x over previous
import jax
import jax.numpy as jnp
from jax import lax
from jax.experimental import pallas as pl
from jax.experimental.pallas import tpu as pltpu
from jax.experimental.pallas import tpu_sc as plsc

B = 16384
ED = 64
RD = 32
NE = 1000000
NW = 32
PASS = 128
NPASS = B // (NW * PASS)
CH = 16
NCH = PASS // CH
NR = PASS * ED // 128


def _sc_body(head_hbm, rel_hbm, tail_hbm, entf_hbm, relemb_hbm, projt_hbm,
             out_hbm, hadr, tadr, hraw, traw, rebuf, obuf, pb0, pb1,
             hidx, tidx, ridx, sem_g, sem_p0, sem_p1):
  wid = lax.axis_index("s") * 2 + lax.axis_index("c")
  iota = lax.iota(jnp.int32, 16)
  kbase = [(iota + c * 16) * NE for c in range(ED // 16)]

  def one_pass(p, carry):
    brow = wid * NPASS + p
    base = brow * PASS

    pltpu.sync_copy(head_hbm.at[pl.ds(brow, 1)], hidx)
    pltpu.sync_copy(tail_hbm.at[pl.ds(brow, 1)], tidx)
    pltpu.sync_copy(rel_hbm.at[pl.ds(brow, 1)], ridx)

    def bld(bb, carry2):
      hv = hidx[0, pl.ds(bb * 16, 16)]
      tv = tidx[0, pl.ds(bb * 16, 16)]
      for j in range(16):
        hs = jnp.broadcast_to(hv[j], (16,))
        ts = jnp.broadcast_to(tv[j], (16,))
        row = bb * 8 + (j >> 1)
        for c in range(ED // 16):
          col = (j & 1) * ED + c * 16
          hadr[row, pl.ds(col, 16)] = kbase[c] + hs
          tadr[row, pl.ds(col, 16)] = kbase[c] + ts
      return carry2
    lax.fori_loop(0, PASS // 16, bld, 0)

    g3 = pltpu.make_async_copy(relemb_hbm.at[ridx.at[0]], rebuf, sem_g)
    g3.start()

    def p_desc(c, buf, sem):
      return pltpu.make_async_copy(
          projt_hbm.at[ridx.at[0, pl.ds(c * CH, CH)]], buf, sem)

    p_desc(0, pb0, sem_p0).start()
    p_desc(1, pb1, sem_p1).start()

    def fire(r, carry2):
      pltpu.make_async_copy(entf_hbm.at[hadr.at[r]], hraw.at[r], sem_g).start()
      pltpu.make_async_copy(entf_hbm.at[tadr.at[r]], traw.at[r], sem_g).start()
      return carry2
    lax.fori_loop(0, NR, fire, 0)

    def drain(r, carry2):
      pltpu.make_async_copy(entf_hbm.at[hadr.at[r]], hraw.at[r], sem_g).wait()
      pltpu.make_async_copy(entf_hbm.at[tadr.at[r]], traw.at[r], sem_g).wait()
      return carry2
    lax.fori_loop(0, NR, drain, 0)
    g3.wait()

    def dsub(i, carry2):
      r = i >> 3
      col = (i & 7) * 16
      hraw[r, pl.ds(col, 16)] = (hraw[r, pl.ds(col, 16)]
                                 - traw[r, pl.ds(col, 16)])
      return carry2
    lax.fori_loop(0, PASS * ED // 16, dsub, 0)

    def ring(it, carry2):
      for bb, (buf, sem) in enumerate(((pb0, sem_p0), (pb1, sem_p1))):
        c = it * 2 + bb
        p_desc(c, buf, sem).wait()

        def triple(s, carry3):
          b = c * CH + s
          dr = b >> 1
          dc = (b & 1) * ED
          a0 = rebuf[b, pl.ds(0, 16)]
          a1 = rebuf[b, pl.ds(16, 16)]
          for kk in range(ED // 16):
            dv = hraw[dr, pl.ds(dc + kk * 16, 16)]
            for j in range(16):
              k = kk * 16 + j
              bc = jnp.broadcast_to(dv[j], (16,))
              p0 = buf[s, pl.ds(k * RD, 16)]
              p1 = buf[s, pl.ds(k * RD + 16, 16)]
              a0 = a0 + bc * p0
              a1 = a1 + bc * p1
          obuf[b, pl.ds(0, 16)] = a0
          obuf[b, pl.ds(16, 16)] = a1
          return carry3
        lax.fori_loop(0, CH, triple, 0)

        nxt = c + 2

        @pl.when(nxt < NCH)
        def _():
          p_desc(nxt, buf, sem).start()
      return carry2
    lax.fori_loop(0, NCH // 2, ring, 0)

    pltpu.sync_copy(obuf, out_hbm.at[pl.ds(base, PASS)])
    return carry
  lax.fori_loop(0, NPASS, one_pass, 0)


def _tc_norm_body(x_ref, o_ref):
  x = x_ref[...]
  o_ref[...] = jnp.sqrt(jnp.sum(x * x, axis=1))


def kernel(head, relation, tail, entity_table, relation_table, proj_table):
  head2 = head.reshape(128, 128).astype(jnp.int32)
  rel2 = relation.reshape(128, 128).astype(jnp.int32)
  tail2 = tail.reshape(128, 128).astype(jnp.int32)
  entf = entity_table.T.reshape(-1)
  projt = proj_table.reshape(-1, RD, ED).transpose(0, 2, 1).reshape(-1, RD * ED)

  sc = pl.kernel(
      _sc_body,
      out_type=jax.ShapeDtypeStruct((B, RD), jnp.float32),
      mesh=plsc.VectorSubcoreMesh(core_axis_name="c", subcore_axis_name="s"),
      compiler_params=pltpu.CompilerParams(use_tc_tiling_on_sc=False),
      scratch_types=[
          pltpu.VMEM((NR, 128), jnp.int32),
          pltpu.VMEM((NR, 128), jnp.int32),
          pltpu.VMEM((NR, 128), jnp.float32),
          pltpu.VMEM((NR, 128), jnp.float32),
          pltpu.VMEM((PASS, RD), jnp.float32),
          pltpu.VMEM((PASS, RD), jnp.float32),
          pltpu.VMEM((CH, RD * ED), jnp.float32),
          pltpu.VMEM((CH, RD * ED), jnp.float32),
          pltpu.VMEM((1, PASS), jnp.int32),
          pltpu.VMEM((1, PASS), jnp.int32),
          pltpu.VMEM((1, PASS), jnp.int32),
          pltpu.SemaphoreType.DMA,
          pltpu.SemaphoreType.DMA,
          pltpu.SemaphoreType.DMA,
      ],
  )
  diff = sc(head2, rel2, tail2, entf, relation_table, projt)

  out = pl.pallas_call(
      _tc_norm_body,
      grid=(16,),
      in_specs=[pl.BlockSpec((B // 16, RD), lambda i: (i, 0))],
      out_specs=pl.BlockSpec((B // 16,), lambda i: (i,)),
      out_shape=jax.ShapeDtypeStruct((B,), jnp.float32),
  )(diff)
  return out


# speedup vs baseline: 6.9949x; 6.9911x over previous
import jax
import jax.numpy as jnp
from jax import lax
from jax.experimental import pallas as pl
from jax.experimental.pallas import tpu as pltpu
from jax.experimental.pallas import tpu_sc as plsc

B = 16384
ED = 64
RD = 32
NW = 32
PW = B // NW
HP = PW // 2
CH = 16
NCH = HP // CH


def _sc_body(head_hbm, rel_hbm, tail_hbm, ent_hbm, relemb_hbm, projt_hbm,
             out_hbm, hbuf, tbuf, rebuf, obuf, pb0, pb1, hidx, tidx, ridx,
             sem_g, sem_p0, sem_p1):
  wid = lax.axis_index("s") * 2 + lax.axis_index("c")
  for half in range(2):
    base = wid * PW + half * HP
    r0 = wid * 4 + half * 2

    pltpu.sync_copy(head_hbm.at[pl.ds(r0, 2)], hidx)
    pltpu.sync_copy(tail_hbm.at[pl.ds(r0, 2)], tidx)
    pltpu.sync_copy(rel_hbm.at[pl.ds(r0, 2)], ridx)

    gathers = []
    for c in range(2):
      gathers.append(pltpu.make_async_copy(
          ent_hbm.at[hidx.at[c]], hbuf.at[pl.ds(c * 128, 128)], sem_g))
      gathers.append(pltpu.make_async_copy(
          ent_hbm.at[tidx.at[c]], tbuf.at[pl.ds(c * 128, 128)], sem_g))
      gathers.append(pltpu.make_async_copy(
          relemb_hbm.at[ridx.at[c]], rebuf.at[pl.ds(c * 128, 128)], sem_g))
    for g in gathers:
      g.start()

    def p_desc(c, buf, sem):
      row = c // 8
      col = (c % 8) * CH
      return pltpu.make_async_copy(
          projt_hbm.at[ridx.at[row, pl.ds(col, CH)]], buf, sem)

    p_desc(0, pb0, sem_p0).start()
    p_desc(1, pb1, sem_p1).start()

    for g in gathers:
      g.wait()

    def dsub(i, carry):
      b = i // 4
      k = (i % 4) * 16
      hbuf[b, pl.ds(k, 16)] = hbuf[b, pl.ds(k, 16)] - tbuf[b, pl.ds(k, 16)]
      return carry
    lax.fori_loop(0, HP * 4, dsub, 0)

    def ring(it, carry):
      for bb, (buf, sem) in enumerate(((pb0, sem_p0), (pb1, sem_p1))):
        c = it * 2 + bb
        p_desc(c, buf, sem).wait()

        def triple(s, carry2):
          b = c * CH + s
          a0 = rebuf[b, pl.ds(0, 16)]
          a1 = rebuf[b, pl.ds(16, 16)]
          for kk in range(ED // 16):
            dv = hbuf[b, pl.ds(kk * 16, 16)]
            for j in range(16):
              k = kk * 16 + j
              bc = jnp.broadcast_to(dv[j], (16,))
              p0 = buf[s, pl.ds(k * RD, 16)]
              p1 = buf[s, pl.ds(k * RD + 16, 16)]
              a0 = a0 + bc * p0
              a1 = a1 + bc * p1
          obuf[b, pl.ds(0, 16)] = a0
          obuf[b, pl.ds(16, 16)] = a1
          return carry2
        lax.fori_loop(0, CH, triple, 0)

        nxt = c + 2

        @pl.when(nxt < NCH)
        def _():
          p_desc(nxt, buf, sem).start()
      return carry
    lax.fori_loop(0, NCH // 2, ring, 0)

    pltpu.sync_copy(obuf, out_hbm.at[pl.ds(base, HP)])


def _tc_norm_body(x_ref, o_ref):
  x = x_ref[...]
  o_ref[...] = jnp.sqrt(jnp.sum(x * x, axis=1))


def kernel(head, relation, tail, entity_table, relation_table, proj_table):
  head2 = head.reshape(128, 128).astype(jnp.int32)
  rel2 = relation.reshape(128, 128).astype(jnp.int32)
  tail2 = tail.reshape(128, 128).astype(jnp.int32)
  projt = proj_table.reshape(-1, RD, ED).transpose(0, 2, 1).reshape(-1, RD * ED)

  sc = pl.kernel(
      _sc_body,
      out_type=jax.ShapeDtypeStruct((B, RD), jnp.float32),
      mesh=plsc.VectorSubcoreMesh(core_axis_name="c", subcore_axis_name="s"),
      compiler_params=pltpu.CompilerParams(use_tc_tiling_on_sc=False),
      scratch_types=[
          pltpu.VMEM((HP, ED), jnp.float32),
          pltpu.VMEM((HP, ED), jnp.float32),
          pltpu.VMEM((HP, RD), jnp.float32),
          pltpu.VMEM((HP, RD), jnp.float32),
          pltpu.VMEM((CH, RD * ED), jnp.float32),
          pltpu.VMEM((CH, RD * ED), jnp.float32),
          pltpu.VMEM((2, 128), jnp.int32),
          pltpu.VMEM((2, 128), jnp.int32),
          pltpu.VMEM((2, 128), jnp.int32),
          pltpu.SemaphoreType.DMA,
          pltpu.SemaphoreType.DMA,
          pltpu.SemaphoreType.DMA,
      ],
  )
  diff = sc(head2, rel2, tail2, entity_table, relation_table, projt)

  out = pl.pallas_call(
      _tc_norm_body,
      grid=(16,),
      in_specs=[pl.BlockSpec((B // 16, RD), lambda i: (i, 0))],
      out_specs=pl.BlockSpec((B // 16,), lambda i: (i,)),
      out_shape=jax.ShapeDtypeStruct((B,), jnp.float32),
  )(diff)
  return out


# speedup vs baseline: 7.3911x; 1.0566x over previous
import jax
import jax.numpy as jnp
from jax import lax
from jax.experimental import pallas as pl
from jax.experimental.pallas import tpu as pltpu
from jax.experimental.pallas import tpu_sc as plsc

B = 16384
ED = 64
RD = 32
NW = 32
PW = B // NW
HP = PW // 2
CH = 16
NCH = HP // CH


def _sc_body(head_hbm, rel_hbm, tail_hbm, ent_hbm, relemb_hbm, projt_hbm,
             out_hbm, hbuf, tbuf, rebuf, obuf, pb0, pb1, hidx, tidx, ridx,
             sem_g, sem_p0, sem_p1):
  wid = lax.axis_index("s") * 2 + lax.axis_index("c")
  for half in range(2):
    base = wid * PW + half * HP
    r0 = wid * 4 + half * 2

    pltpu.sync_copy(head_hbm.at[pl.ds(r0, 2)], hidx)
    pltpu.sync_copy(tail_hbm.at[pl.ds(r0, 2)], tidx)
    pltpu.sync_copy(rel_hbm.at[pl.ds(r0, 2)], ridx)

    gathers = []
    for c in range(2):
      gathers.append(pltpu.make_async_copy(
          ent_hbm.at[hidx.at[c]], hbuf.at[pl.ds(c * 128, 128)], sem_g))
      gathers.append(pltpu.make_async_copy(
          ent_hbm.at[tidx.at[c]], tbuf.at[pl.ds(c * 128, 128)], sem_g))
      gathers.append(pltpu.make_async_copy(
          relemb_hbm.at[ridx.at[c]], rebuf.at[pl.ds(c * 128, 128)], sem_g))
    for g in gathers:
      g.start()

    def p_desc(c, buf, sem):
      row = c // 8
      col = (c % 8) * CH
      return pltpu.make_async_copy(
          projt_hbm.at[ridx.at[row, pl.ds(col, CH)]], buf, sem)

    p_desc(0, pb0, sem_p0).start()
    p_desc(1, pb1, sem_p1).start()

    for g in gathers:
      g.wait()

    def dsub(i, carry):
      b = i // 4
      k = (i % 4) * 16
      hbuf[b, pl.ds(k, 16)] = hbuf[b, pl.ds(k, 16)] - tbuf[b, pl.ds(k, 16)]
      return carry
    lax.fori_loop(0, HP * 4, dsub, 0)

    def ring(it, carry):
      for bb, (buf, sem) in enumerate(((pb0, sem_p0), (pb1, sem_p1))):
        c = it * 2 + bb
        p_desc(c, buf, sem).wait()

        def triple(s, carry2):
          b = c * CH + s
          a0 = rebuf[b, pl.ds(0, 16)]
          a1 = rebuf[b, pl.ds(16, 16)]
          for kk in range(ED // 16):
            dv = hbuf[b, pl.ds(kk * 16, 16)]
            for j in range(16):
              k = kk * 16 + j
              bc = jnp.broadcast_to(dv[j], (16,))
              pv = buf[s, pl.ds(k * RD, RD)]
              p0, p1 = plsc.unpack(pv, format=plsc.PackFormat.INTERLEAVED)
              a0 = a0 + bc * p0
              a1 = a1 + bc * p1
          obuf[b, pl.ds(0, 16)] = a0
          obuf[b, pl.ds(16, 16)] = a1
          return carry2
        lax.fori_loop(0, CH, triple, 0)

        nxt = c + 2

        @pl.when(nxt < NCH)
        def _():
          p_desc(nxt, buf, sem).start()
      return carry
    lax.fori_loop(0, NCH // 2, ring, 0)

    pltpu.sync_copy(obuf, out_hbm.at[pl.ds(base, HP)])


def _tc_norm_body(x_ref, o_ref):
  x = x_ref[...]
  o_ref[...] = jnp.sqrt(jnp.sum(x * x, axis=1))


def kernel(head, relation, tail, entity_table, relation_table, proj_table):
  head2 = head.reshape(128, 128).astype(jnp.int32)
  rel2 = relation.reshape(128, 128).astype(jnp.int32)
  tail2 = tail.reshape(128, 128).astype(jnp.int32)
  projt = (proj_table.reshape(-1, RD, ED).transpose(0, 2, 1)
           .reshape(-1, ED, 2, 16).transpose(0, 1, 3, 2)
           .reshape(-1, RD * ED).astype(jnp.bfloat16))

  sc = pl.kernel(
      _sc_body,
      out_type=jax.ShapeDtypeStruct((B, RD), jnp.float32),
      mesh=plsc.VectorSubcoreMesh(core_axis_name="c", subcore_axis_name="s"),
      compiler_params=pltpu.CompilerParams(use_tc_tiling_on_sc=False,
                                           needs_layout_passes=False),
      scratch_types=[
          pltpu.VMEM((HP, ED), jnp.float32),
          pltpu.VMEM((HP, ED), jnp.float32),
          pltpu.VMEM((HP, RD), jnp.float32),
          pltpu.VMEM((HP, RD), jnp.float32),
          pltpu.VMEM((CH, RD * ED), jnp.bfloat16),
          pltpu.VMEM((CH, RD * ED), jnp.bfloat16),
          pltpu.VMEM((2, 128), jnp.int32),
          pltpu.VMEM((2, 128), jnp.int32),
          pltpu.VMEM((2, 128), jnp.int32),
          pltpu.SemaphoreType.DMA,
          pltpu.SemaphoreType.DMA,
          pltpu.SemaphoreType.DMA,
      ],
  )
  diff = sc(head2, rel2, tail2, entity_table, relation_table, projt)

  out = pl.pallas_call(
      _tc_norm_body,
      grid=(16,),
      in_specs=[pl.BlockSpec((B // 16, RD), lambda i: (i, 0))],
      out_specs=pl.BlockSpec((B // 16,), lambda i: (i,)),
      out_shape=jax.ShapeDtypeStruct((B,), jnp.float32),
  )(diff)
  return out
